# baseline (speedup 1.0000x reference)
import functools

import jax
import jax.numpy as jnp
from jax import lax
from jax.experimental import pallas as pl
from jax.experimental.pallas import tpu as pltpu
from jax.experimental.pallas import tpu_sc as plsc

_LANES = 16
_WIN = 128
_CAP = 640
_BCAP = 16
_DEPTH = 6
_SENT = 16384


def _scalar(x):
    return lax.reduce_max(x, axes=(0,))


def kernel(labels, embedding_table):
    info = plsc.get_sparse_core_info()
    nw = info.num_cores * info.num_subcores
    (batch,) = labels.shape
    num_rows, dim = embedding_table.shape
    labels = labels.astype(jnp.int32)
    table_t = embedding_table.T

    n_tc = (num_rows + _WIN - 1) // _WIN
    tc_per_w = ((n_tc + nw - 1) // nw + 1) // 2 * 2
    last_tc = n_tc - 1
    last_w = num_rows - last_tc * _WIN
    tail_t = jnp.pad(
        lax.slice(table_t, (0, last_tc * _WIN), (dim, num_rows)),
        ((0, 0), (0, _WIN - last_w)))

    mesh = plsc.VectorSubcoreMesh(core_axis_name="c", subcore_axis_name="s")

    @functools.partial(
        pl.kernel,
        mesh=mesh,
        out_type=(
            jax.ShapeDtypeStruct((nw * _CAP * dim,), jnp.float32),
            jax.ShapeDtypeStruct((nw * _CAP,), jnp.int32),
            jax.ShapeDtypeStruct((nw * 8 + 8,), jnp.int32),
        ),
        compiler_params=pltpu.CompilerParams(needs_layout_passes=False),
        scratch_types=[
            pltpu.VMEM((512,), jnp.int32),
            pltpu.VMEM((_CAP + _LANES,), jnp.int32),
            pltpu.VMEM((_CAP + _LANES,), jnp.int32),
            pltpu.VMEM((_CAP + _LANES,), jnp.int32),
            pltpu.VMEM((_DEPTH, dim, _WIN), jnp.float32),
            pltpu.VMEM((2 * 64 * dim,), jnp.float32),
            pltpu.VMEM((tc_per_w + 2, _BCAP), jnp.int32),
            pltpu.VMEM((tc_per_w + 2 + _LANES, ), jnp.int32),
            pltpu.VMEM((tc_per_w + 2 + _LANES, ), jnp.int32),
            pltpu.SMEM((8,), jnp.int32),
        ] + [pltpu.SemaphoreType.DMA] * _DEPTH,
    )
    def scan_kernel(idx_hbm, table_hbm, tail_hbm, rows_hbm, pos_hbm,
                    cnt_hbm, idx_v, own_r, own_b, own_b2, win_v, rowbuf,
                    bucket, counts, newin, scell, *sems):
        wid = lax.axis_index("s") * info.num_cores + lax.axis_index("c")
        tc0 = wid * tc_per_w
        lo_w = tc0 * _WIN
        hi_w = (tc0 + tc_per_w) * _WIN
        lane = lax.iota(jnp.int32, _LANES)
        zeros = jnp.zeros((_LANES,), jnp.int32)
        def zcount(k, c):
            counts[pl.ds(k * _LANES, _LANES)] = zeros
            return c

        lax.fori_loop(0, (tc_per_w + 2 + _LANES) // _LANES, zcount, 0)

        def stage_body(st, cnt):
            pltpu.sync_copy(idx_hbm.at[pl.ds(st * 512, 512)], idx_v)

            def bin_body(v, cnt):
                vec = idx_v[pl.ds(v * _LANES, _LANES)]
                m = (vec >= lo_w) & (vec < hi_w)
                bvec = lane + st * 512 + v * _LANES
                plsc.store_compressed(own_r.at[pl.ds(cnt, _LANES)], vec,
                                      mask=m)
                plsc.store_compressed(own_b.at[pl.ds(cnt, _LANES)], bvec,
                                      mask=m)
                k = plsc.all_reduce_population_count(m)[0]
                return jnp.minimum(cnt + k, _CAP)

            return lax.fori_loop(0, 512 // _LANES, bin_body, cnt)

        cnt = lax.fori_loop(0, batch // 512, stage_body, jnp.int32(0))

        lane0 = lane == 0

        def buck_body(q, c):
            r = own_r[pl.ds(q, _LANES)][0]
            t = (r - lo_w) >> 7
            j = r & (_WIN - 1)
            bc = counts[pl.ds(t, _LANES)][0]

            @pl.when(bc < _BCAP)
            def _():
                e = (j << 12) | q
                plsc.store_scatter(
                    bucket,
                    [jnp.full((_LANES,), t, jnp.int32),
                     jnp.full((_LANES,), bc, jnp.int32)],
                    jnp.full((_LANES,), e, jnp.int32), mask=lane0)
                plsc.store_scatter(
                    counts, [jnp.full((_LANES,), t, jnp.int32)],
                    jnp.full((_LANES,), bc + 1, jnp.int32), mask=lane0)

            return c

        lax.fori_loop(0, cnt, buck_body, 0)

        def ne_body(t, n):
            bc = counts[pl.ds(t, _LANES)][0]

            @pl.when(bc > 0)
            def _():
                plsc.store_scatter(
                    newin, [jnp.full((_LANES,), n, jnp.int32)],
                    jnp.full((_LANES,), t, jnp.int32), mask=lane0)

            return n + jnp.where(bc > 0, 1, 0).astype(jnp.int32)

        n_ne = lax.fori_loop(0, tc_per_w, ne_body, jnp.int32(0))

        def fetch(i, slot):
            @pl.when(i < n_ne)
            def _():
                t = newin[pl.ds(i, _LANES)][0]
                tc = tc0 + t

                @pl.when(tc < last_tc)
                def _():
                    off = pl.multiple_of(tc * _WIN, _WIN)
                    pltpu.async_copy(table_hbm.at[:, pl.ds(off, _WIN)],
                                     win_v.at[slot], sems[slot])

                @pl.when(tc == last_tc)
                def _():
                    pltpu.async_copy(tail_hbm, win_v.at[slot], sems[slot])

        def wait(i, slot):
            @pl.when(i < n_ne)
            def _():
                t = newin[pl.ds(i, _LANES)][0]
                tc = tc0 + t

                @pl.when(tc < last_tc)
                def _():
                    off = pl.multiple_of(tc * _WIN, _WIN)
                    pltpu.make_async_copy(table_hbm.at[:, pl.ds(off, _WIN)],
                                          win_v.at[slot], sems[slot]).wait()

                @pl.when(tc == last_tc)
                def _():
                    pltpu.make_async_copy(tail_hbm, win_v.at[slot],
                                          sems[slot]).wait()

        def extract(i, slot):
            @pl.when(i < n_ne)
            def _():
                t = newin[pl.ds(i, _LANES)][0]
                bc = counts[pl.ds(t, _LANES)][0]
                erow = bucket[t, pl.ds(0, _BCAP)]

                for q2 in range(_BCAP):
                    @pl.when(q2 < bc)
                    def _():
                        e = erow[q2]
                        j = e >> 12
                        q = e & 4095
                        jv = jnp.full((_LANES,), j, jnp.int32)
                        o = scell[0]
                        ob = (o & 127) * dim
                        for k in range(dim // _LANES):
                            cvec = lane + k * _LANES
                            vals = plsc.load_gather(win_v.at[slot],
                                                    [cvec, jv])
                            rowbuf[pl.ds(ob + k * _LANES, _LANES)] = vals
                        vecb = own_b[pl.ds(q, _LANES)]
                        plsc.store_scatter(
                            own_b2, [jnp.full((_LANES,), o, jnp.int32)],
                            vecb, mask=lane0)
                        scell[0] = o + 1

                o = scell[0]
                fl = scell[1]

                @pl.when(o - fl >= 64)
                def _():
                    half = ((fl >> 6) & 1) * (64 * dim)
                    dst = pl.multiple_of((wid * _CAP + fl) * dim, 64)
                    pltpu.sync_copy(rowbuf.at[pl.ds(half, 64 * dim)],
                                    rows_hbm.at[pl.ds(dst, 64 * dim)])
                    scell[1] = fl + 64

        scell[0] = jnp.int32(0)
        scell[1] = jnp.int32(0)

        for s in range(_DEPTH - 1):
            fetch(jnp.int32(s), s)

        n_blk = (tc_per_w + _DEPTH - 1) // _DEPTH

        def blk_body(blk, c):
            i0 = blk * _DEPTH
            for s in range(_DEPTH):
                i = i0 + s
                wait(i, s)
                extract(i, s)
                fetch(i + _DEPTH - 1, (s + _DEPTH - 1) % _DEPTH)
            return c

        lax.fori_loop(0, n_blk, blk_body, 0)

        o_real = scell[0]
        o_pad = (o_real + 63) & ~63

        @pl.when(o_real > 0)
        def _():
            lrow = (o_real - 1) & 127
            lastb = own_b2[pl.ds(o_real - 1, _LANES)]

            def pad_body(sl, c):
                for k in range(dim // _LANES):
                    rowbuf[pl.ds((sl & 127) * dim + k * _LANES, _LANES)] = (
                        rowbuf[pl.ds(lrow * dim + k * _LANES, _LANES)])
                plsc.store_scatter(
                    own_b2, [jnp.full((_LANES,), sl, jnp.int32)],
                    lastb, mask=lane0)
                return c

            lax.fori_loop(o_real, o_pad, pad_body, 0)
            scell[0] = o_pad

        cw = jnp.full((_LANES,), 0, jnp.int32)
        plsc.store_scatter(own_r, [lane], cw, mask=lane < 8)
        plsc.store_scatter(own_r, [lane], jnp.full((_LANES,), o_pad // 64,
                                                   jnp.int32), mask=lane0)
        pltpu.sync_copy(own_r.at[pl.ds(0, 8)],
                        cnt_hbm.at[pl.ds(wid * 8, 8)])

        def drain(d, c):
            o = scell[0]
            fl = scell[1]

            @pl.when(fl < o)
            def _():
                half = ((fl >> 6) & 1) * (64 * dim)
                dst = pl.multiple_of((wid * _CAP + fl) * dim, 64)
                pltpu.sync_copy(rowbuf.at[pl.ds(half, 64 * dim)],
                                rows_hbm.at[pl.ds(dst, 64 * dim)])
                scell[1] = fl + 64

            return c

        lax.fori_loop(0, _CAP // 64 + 1, drain, 0)

        pltpu.sync_copy(own_b2.at[pl.ds(0, _CAP)],
                        pos_hbm.at[pl.ds(wid * _CAP, _CAP)])

    rows_flat, b_list, cnts = scan_kernel(labels, table_t, tail_t)
    rows = rows_flat.reshape(nw * _CAP, dim)
    idx2 = b_list.reshape(nw * (_CAP // 64), 64)
    n_chunk = _CAP // 64

    @functools.partial(
        pl.kernel,
        mesh=mesh,
        out_type=jax.ShapeDtypeStruct((batch, dim), jnp.float32),
        compiler_params=pltpu.CompilerParams(use_tc_tiling_on_sc=False),
        scratch_types=[
            pltpu.VMEM((n_chunk, 64), jnp.int32),
            pltpu.VMEM((_LANES,), jnp.int32),
            pltpu.VMEM((_CAP, dim), jnp.float32),
            pltpu.SemaphoreType.DMA,
        ],
    )
    def permute_kernel(pos_hbm, rows_hbm, cnt_hbm, out_hbm,
                       idx_v, cbuf, rows_v, sem):
        wid = lax.axis_index("s") * info.num_cores + lax.axis_index("c")
        base = wid * _CAP
        pltpu.sync_copy(pos_hbm.at[pl.ds(wid * n_chunk, n_chunk), :], idx_v)
        pltpu.sync_copy(cnt_hbm.at[pl.ds(wid * 8, _LANES)], cbuf)
        nch = cbuf[pl.ds(0, _LANES)][0]
        pltpu.sync_copy(rows_hbm.at[pl.ds(base, _CAP), :], rows_v)
        for k in range(n_chunk):
            @pl.when(k < nch)
            def _():
                pltpu.async_copy(rows_v.at[pl.ds(k * 64, 64), :],
                                 out_hbm.at[idx_v.at[k]], sem)
        for k in range(n_chunk):
            @pl.when(k < nch)
            def _():
                pltpu.make_async_copy(rows_v.at[pl.ds(k * 64, 64), :],
                                      out_hbm.at[idx_v.at[k]], sem).wait()

    return permute_kernel(idx2, rows, cnts)


# speedup vs baseline: 1.5258x; 1.5258x over previous
import functools

import jax
import jax.numpy as jnp
from jax import lax
from jax.experimental import pallas as pl
from jax.experimental.pallas import tpu as pltpu
from jax.experimental.pallas import tpu_sc as plsc

_LANES = 16
_WIN = 128
_CAP = 640
_BCAP = 16
_DEPTH = 6
_SENT = 16384


def _scalar(x):
    return lax.reduce_max(x, axes=(0,))


def kernel(labels, embedding_table):
    info = plsc.get_sparse_core_info()
    nw = info.num_cores * info.num_subcores
    (batch,) = labels.shape
    num_rows, dim = embedding_table.shape
    labels = labels.astype(jnp.int32)
    table_t = embedding_table.T

    n_tc = (num_rows + _WIN - 1) // _WIN
    tc_per_w = ((n_tc + nw - 1) // nw + 1) // 2 * 2
    last_tc = n_tc - 1
    last_w = num_rows - last_tc * _WIN
    tail_t = jnp.pad(
        lax.slice(table_t, (0, last_tc * _WIN), (dim, num_rows)),
        ((0, 0), (0, _WIN - last_w)))

    mesh = plsc.VectorSubcoreMesh(core_axis_name="c", subcore_axis_name="s")

    @functools.partial(
        pl.kernel,
        mesh=mesh,
        out_type=(
            jax.ShapeDtypeStruct((nw * _CAP * dim,), jnp.float32),
            jax.ShapeDtypeStruct((nw * _CAP,), jnp.int32),
            jax.ShapeDtypeStruct((nw * 8 + 8,), jnp.int32),
        ),
        compiler_params=pltpu.CompilerParams(needs_layout_passes=False),
        scratch_types=[
            pltpu.VMEM((512,), jnp.int32),
            pltpu.VMEM((_CAP + _LANES,), jnp.int32),
            pltpu.VMEM((_CAP + _LANES,), jnp.int32),
            pltpu.VMEM((_CAP + _LANES,), jnp.int32),
            pltpu.VMEM((_DEPTH, dim, _WIN), jnp.float32),
            pltpu.VMEM((2 * 64 * dim,), jnp.float32),
            pltpu.VMEM((tc_per_w + 2, _BCAP), jnp.int32),
            pltpu.VMEM((tc_per_w + 2 + _LANES, ), jnp.int32),
            pltpu.VMEM((tc_per_w + 2 + _LANES, ), jnp.int32),
            pltpu.SMEM((8,), jnp.int32),
        ] + [pltpu.SemaphoreType.DMA] * _DEPTH,
    )
    def scan_kernel(idx_hbm, table_hbm, tail_hbm, rows_hbm, pos_hbm,
                    cnt_hbm, idx_v, own_r, own_b, own_b2, win_v, rowbuf,
                    bucket, counts, newin, scell, *sems):
        wid = lax.axis_index("s") * info.num_cores + lax.axis_index("c")
        tc0 = wid * tc_per_w
        lo_w = tc0 * _WIN
        hi_w = (tc0 + tc_per_w) * _WIN
        lane = lax.iota(jnp.int32, _LANES)
        zeros = jnp.zeros((_LANES,), jnp.int32)
        def zcount(k, c):
            counts[pl.ds(k * _LANES, _LANES)] = zeros
            return c

        lax.fori_loop(0, (tc_per_w + 2 + _LANES) // _LANES, zcount, 0)

        def stage_body(st, cnt):
            pltpu.sync_copy(idx_hbm.at[pl.ds(st * 512, 512)], idx_v)

            def bin_body(v, cnt):
                vec = idx_v[pl.ds(v * _LANES, _LANES)]
                m = (vec >= lo_w) & (vec < hi_w)
                bvec = lane + st * 512 + v * _LANES
                plsc.store_compressed(own_r.at[pl.ds(cnt, _LANES)], vec,
                                      mask=m)
                plsc.store_compressed(own_b.at[pl.ds(cnt, _LANES)], bvec,
                                      mask=m)
                k = plsc.all_reduce_population_count(m)[0]
                return jnp.minimum(cnt + k, _CAP)

            return lax.fori_loop(0, 512 // _LANES, bin_body, cnt)

        cnt = lax.fori_loop(0, batch // 512, stage_body, jnp.int32(0))

        lane0 = lane == 0

        def buck_body(q, c):
            r = own_r[pl.ds(q, _LANES)][0]
            t = (r - lo_w) >> 7
            j = r & (_WIN - 1)
            bc = counts[pl.ds(t, _LANES)][0]

            @pl.when(bc < _BCAP)
            def _():
                e = (j << 12) | q
                plsc.store_scatter(
                    bucket,
                    [jnp.full((_LANES,), t, jnp.int32),
                     jnp.full((_LANES,), bc, jnp.int32)],
                    jnp.full((_LANES,), e, jnp.int32), mask=lane0)
                plsc.store_scatter(
                    counts, [jnp.full((_LANES,), t, jnp.int32)],
                    jnp.full((_LANES,), bc + 1, jnp.int32), mask=lane0)

            return c

        lax.fori_loop(0, cnt, buck_body, 0)

        def ne_body(t, n):
            bc = counts[pl.ds(t, _LANES)][0]

            @pl.when(bc > 0)
            def _():
                plsc.store_scatter(
                    newin, [jnp.full((_LANES,), n, jnp.int32)],
                    jnp.full((_LANES,), t, jnp.int32), mask=lane0)

            return n + jnp.where(bc > 0, 1, 0).astype(jnp.int32)

        n_ne = lax.fori_loop(0, tc_per_w, ne_body, jnp.int32(0))

        def fetch(i, slot):
            @pl.when(i < n_ne)
            def _():
                t = newin[pl.ds(i, _LANES)][0]
                tc = tc0 + t

                @pl.when(tc < last_tc)
                def _():
                    off = pl.multiple_of(tc * _WIN, _WIN)
                    pltpu.async_copy(table_hbm.at[:, pl.ds(off, _WIN)],
                                     win_v.at[slot], sems[slot])

                @pl.when(tc == last_tc)
                def _():
                    pltpu.async_copy(tail_hbm, win_v.at[slot], sems[slot])

        def wait(i, slot):
            @pl.when(i < n_ne)
            def _():
                t = newin[pl.ds(i, _LANES)][0]
                tc = tc0 + t

                @pl.when(tc < last_tc)
                def _():
                    off = pl.multiple_of(tc * _WIN, _WIN)
                    pltpu.make_async_copy(table_hbm.at[:, pl.ds(off, _WIN)],
                                          win_v.at[slot], sems[slot]).wait()

                @pl.when(tc == last_tc)
                def _():
                    pltpu.make_async_copy(tail_hbm, win_v.at[slot],
                                          sems[slot]).wait()

        def extract(i, slot):
            @pl.when(i < n_ne)
            def _():
                t = newin[pl.ds(i, _LANES)][0]
                bc = counts[pl.ds(t, _LANES)][0]
                erow = bucket[t, pl.ds(0, _BCAP)]

                def lab_body(q2, c):
                    e = _scalar(jnp.where(lane == q2, erow, 0))
                    j = e >> 12
                    q = e & 4095
                    jv = jnp.full((_LANES,), j, jnp.int32)
                    o = scell[0]
                    ob = (o & 127) * dim
                    for k in range(dim // _LANES):
                        cvec = lane + k * _LANES
                        vals = plsc.load_gather(win_v.at[slot], [cvec, jv])
                        rowbuf[pl.ds(ob + k * _LANES, _LANES)] = vals
                    vecb = own_b[pl.ds(q, _LANES)]
                    plsc.store_scatter(
                        own_b2, [jnp.full((_LANES,), o, jnp.int32)],
                        vecb, mask=lane0)
                    scell[0] = o + 1
                    return c

                lax.fori_loop(0, bc, lab_body, 0)

                o = scell[0]
                fl = scell[1]

                @pl.when(o - fl >= 64)
                def _():
                    half = ((fl >> 6) & 1) * (64 * dim)
                    dst = pl.multiple_of((wid * _CAP + fl) * dim, 64)
                    pltpu.sync_copy(rowbuf.at[pl.ds(half, 64 * dim)],
                                    rows_hbm.at[pl.ds(dst, 64 * dim)])
                    scell[1] = fl + 64

        scell[0] = jnp.int32(0)
        scell[1] = jnp.int32(0)

        for s in range(_DEPTH - 1):
            fetch(jnp.int32(s), s)

        n_blk = (tc_per_w + _DEPTH - 1) // _DEPTH

        def blk_body(blk, c):
            i0 = blk * _DEPTH
            for s in range(_DEPTH):
                i = i0 + s
                wait(i, s)
                extract(i, s)
                fetch(i + _DEPTH - 1, (s + _DEPTH - 1) % _DEPTH)
            return c

        lax.fori_loop(0, n_blk, blk_body, 0)

        o_real = scell[0]
        o_pad = (o_real + 63) & ~63

        @pl.when(o_real > 0)
        def _():
            lrow = (o_real - 1) & 127
            lastb = own_b2[pl.ds(o_real - 1, _LANES)]

            def pad_body(sl, c):
                for k in range(dim // _LANES):
                    rowbuf[pl.ds((sl & 127) * dim + k * _LANES, _LANES)] = (
                        rowbuf[pl.ds(lrow * dim + k * _LANES, _LANES)])
                plsc.store_scatter(
                    own_b2, [jnp.full((_LANES,), sl, jnp.int32)],
                    lastb, mask=lane0)
                return c

            lax.fori_loop(o_real, o_pad, pad_body, 0)
            scell[0] = o_pad

        cw = jnp.full((_LANES,), 0, jnp.int32)
        plsc.store_scatter(own_r, [lane], cw, mask=lane < 8)
        plsc.store_scatter(own_r, [lane], jnp.full((_LANES,), o_pad // 64,
                                                   jnp.int32), mask=lane0)
        pltpu.sync_copy(own_r.at[pl.ds(0, 8)],
                        cnt_hbm.at[pl.ds(wid * 8, 8)])

        def drain(d, c):
            o = scell[0]
            fl = scell[1]

            @pl.when(fl < o)
            def _():
                half = ((fl >> 6) & 1) * (64 * dim)
                dst = pl.multiple_of((wid * _CAP + fl) * dim, 64)
                pltpu.sync_copy(rowbuf.at[pl.ds(half, 64 * dim)],
                                rows_hbm.at[pl.ds(dst, 64 * dim)])
                scell[1] = fl + 64

            return c

        lax.fori_loop(0, _CAP // 64 + 1, drain, 0)

        pltpu.sync_copy(own_b2.at[pl.ds(0, _CAP)],
                        pos_hbm.at[pl.ds(wid * _CAP, _CAP)])

    rows_flat, b_list, cnts = scan_kernel(labels, table_t, tail_t)
    rows = rows_flat.reshape(nw * _CAP, dim)
    idx2 = b_list.reshape(nw * (_CAP // 64), 64)
    n_chunk = _CAP // 64

    @functools.partial(
        pl.kernel,
        mesh=mesh,
        out_type=jax.ShapeDtypeStruct((batch, dim), jnp.float32),
        compiler_params=pltpu.CompilerParams(use_tc_tiling_on_sc=False),
        scratch_types=[
            pltpu.VMEM((n_chunk, 64), jnp.int32),
            pltpu.VMEM((_LANES,), jnp.int32),
            pltpu.VMEM((_CAP, dim), jnp.float32),
            pltpu.SemaphoreType.DMA,
        ],
    )
    def permute_kernel(pos_hbm, rows_hbm, cnt_hbm, out_hbm,
                       idx_v, cbuf, rows_v, sem):
        wid = lax.axis_index("s") * info.num_cores + lax.axis_index("c")
        base = wid * _CAP
        pltpu.sync_copy(pos_hbm.at[pl.ds(wid * n_chunk, n_chunk), :], idx_v)
        pltpu.sync_copy(cnt_hbm.at[pl.ds(wid * 8, _LANES)], cbuf)
        nch = cbuf[pl.ds(0, _LANES)][0]
        pltpu.sync_copy(rows_hbm.at[pl.ds(base, _CAP), :], rows_v)
        for k in range(n_chunk):
            @pl.when(k < nch)
            def _():
                pltpu.async_copy(rows_v.at[pl.ds(k * 64, 64), :],
                                 out_hbm.at[idx_v.at[k]], sem)
        for k in range(n_chunk):
            @pl.when(k < nch)
            def _():
                pltpu.make_async_copy(rows_v.at[pl.ds(k * 64, 64), :],
                                      out_hbm.at[idx_v.at[k]], sem).wait()

    return permute_kernel(idx2, rows, cnts)


# speedup vs baseline: 1.6152x; 1.0586x over previous
import functools

import jax
import jax.numpy as jnp
from jax import lax
from jax.experimental import pallas as pl
from jax.experimental.pallas import tpu as pltpu
from jax.experimental.pallas import tpu_sc as plsc

_LANES = 16
_WIN = 128
_CAP = 640
_BCAP = 16
_DEPTH = 6
_SENT = 16384


def _scalar(x):
    return lax.reduce_max(x, axes=(0,))


def kernel(labels, embedding_table):
    info = plsc.get_sparse_core_info()
    nw = info.num_cores * info.num_subcores
    (batch,) = labels.shape
    num_rows, dim = embedding_table.shape
    labels = labels.astype(jnp.int32)
    table_t = embedding_table.T

    n_tc = (num_rows + _WIN - 1) // _WIN
    tc_per_w = ((n_tc + nw - 1) // nw + 1) // 2 * 2
    last_tc = n_tc - 1
    last_w = num_rows - last_tc * _WIN
    tail_t = jnp.pad(
        lax.slice(table_t, (0, last_tc * _WIN), (dim, num_rows)),
        ((0, 0), (0, _WIN - last_w)))

    mesh = plsc.VectorSubcoreMesh(core_axis_name="c", subcore_axis_name="s")

    @functools.partial(
        pl.kernel,
        mesh=mesh,
        out_type=(
            jax.ShapeDtypeStruct((nw * _CAP * dim,), jnp.float32),
            jax.ShapeDtypeStruct((nw * _CAP,), jnp.int32),
            jax.ShapeDtypeStruct((nw * 8 + 8,), jnp.int32),
        ),
        compiler_params=pltpu.CompilerParams(needs_layout_passes=False),
        scratch_types=[
            pltpu.VMEM((1024,), jnp.int32),
            pltpu.VMEM((_CAP + _LANES,), jnp.int32),
            pltpu.VMEM((_CAP + _LANES,), jnp.int32),
            pltpu.VMEM((_CAP + _LANES,), jnp.int32),
            pltpu.VMEM((_DEPTH, dim, _WIN), jnp.float32),
            pltpu.VMEM((2 * 64 * dim,), jnp.float32),
            pltpu.VMEM((tc_per_w + 2, _BCAP), jnp.int32),
            pltpu.VMEM((tc_per_w + 2 + _LANES, ), jnp.int32),
            pltpu.VMEM((tc_per_w + 2 + _LANES, ), jnp.int32),
            pltpu.SMEM((8,), jnp.int32),
        ] + [pltpu.SemaphoreType.DMA] * _DEPTH,
    )
    def scan_kernel(idx_hbm, table_hbm, tail_hbm, rows_hbm, pos_hbm,
                    cnt_hbm, idx_v, own_r, own_b, own_b2, win_v, rowbuf,
                    bucket, counts, newin, scell, *sems):
        wid = lax.axis_index("s") * info.num_cores + lax.axis_index("c")
        tc0 = wid * tc_per_w
        lo_w = tc0 * _WIN
        hi_w = (tc0 + tc_per_w) * _WIN
        lane = lax.iota(jnp.int32, _LANES)
        zeros = jnp.zeros((_LANES,), jnp.int32)
        def zcount(k, c):
            counts[pl.ds(k * _LANES, _LANES)] = zeros
            return c

        lax.fori_loop(0, (tc_per_w + 2 + _LANES) // _LANES, zcount, 0)

        n_stage = batch // 512

        def stage_fetch(st, half):
            @pl.when(st < n_stage)
            def _():
                pltpu.async_copy(idx_hbm.at[pl.ds(st * 512, 512)],
                                 idx_v.at[pl.ds(half * 512, 512)],
                                 sems[half])

        def stage_wait(st, half):
            @pl.when(st < n_stage)
            def _():
                pltpu.make_async_copy(idx_hbm.at[pl.ds(st * 512, 512)],
                                      idx_v.at[pl.ds(half * 512, 512)],
                                      sems[half]).wait()

        def stage_bin(st, half, cnt):
            def bin_body(v, cnt):
                vec = idx_v[pl.ds(half * 512 + v * _LANES, _LANES)]
                m = (vec >= lo_w) & (vec < hi_w)
                bvec = lane + st * 512 + v * _LANES
                plsc.store_compressed(own_r.at[pl.ds(cnt, _LANES)], vec,
                                      mask=m)
                plsc.store_compressed(own_b.at[pl.ds(cnt, _LANES)], bvec,
                                      mask=m)
                k = plsc.all_reduce_population_count(m)[0]
                return jnp.minimum(cnt + k, _CAP)

            return lax.fori_loop(0, 512 // _LANES, bin_body, cnt)

        stage_fetch(jnp.int32(0), 0)

        def stagepair_body(sp, cnt):
            s0 = sp * 2
            stage_wait(s0, 0)
            stage_fetch(s0 + 1, 1)
            cnt = stage_bin(s0, 0, cnt)
            stage_wait(s0 + 1, 1)
            stage_fetch(s0 + 2, 0)
            cnt = stage_bin(s0 + 1, 1, cnt)
            return cnt

        cnt = lax.fori_loop(0, n_stage // 2, stagepair_body, jnp.int32(0))

        lane0 = lane == 0

        def buck_body(q, c):
            r = own_r[pl.ds(q, _LANES)][0]
            t = (r - lo_w) >> 7
            j = r & (_WIN - 1)
            bc = counts[pl.ds(t, _LANES)][0]

            @pl.when(bc < _BCAP)
            def _():
                e = (j << 12) | q
                plsc.store_scatter(
                    bucket,
                    [jnp.full((_LANES,), t, jnp.int32),
                     jnp.full((_LANES,), bc, jnp.int32)],
                    jnp.full((_LANES,), e, jnp.int32), mask=lane0)
                plsc.store_scatter(
                    counts, [jnp.full((_LANES,), t, jnp.int32)],
                    jnp.full((_LANES,), bc + 1, jnp.int32), mask=lane0)

            return c

        lax.fori_loop(0, cnt, buck_body, 0)

        def ne_body(t, n):
            bc = counts[pl.ds(t, _LANES)][0]

            @pl.when(bc > 0)
            def _():
                plsc.store_scatter(
                    newin, [jnp.full((_LANES,), n, jnp.int32)],
                    jnp.full((_LANES,), t, jnp.int32), mask=lane0)

            return n + jnp.where(bc > 0, 1, 0).astype(jnp.int32)

        n_ne = lax.fori_loop(0, tc_per_w, ne_body, jnp.int32(0))

        def fetch(i, slot):
            @pl.when(i < n_ne)
            def _():
                t = newin[pl.ds(i, _LANES)][0]
                tc = tc0 + t

                @pl.when(tc < last_tc)
                def _():
                    off = pl.multiple_of(tc * _WIN, _WIN)
                    pltpu.async_copy(table_hbm.at[:, pl.ds(off, _WIN)],
                                     win_v.at[slot], sems[slot])

                @pl.when(tc == last_tc)
                def _():
                    pltpu.async_copy(tail_hbm, win_v.at[slot], sems[slot])

        def wait(i, slot):
            @pl.when(i < n_ne)
            def _():
                t = newin[pl.ds(i, _LANES)][0]
                tc = tc0 + t

                @pl.when(tc < last_tc)
                def _():
                    off = pl.multiple_of(tc * _WIN, _WIN)
                    pltpu.make_async_copy(table_hbm.at[:, pl.ds(off, _WIN)],
                                          win_v.at[slot], sems[slot]).wait()

                @pl.when(tc == last_tc)
                def _():
                    pltpu.make_async_copy(tail_hbm, win_v.at[slot],
                                          sems[slot]).wait()

        def extract(i, slot):
            @pl.when(i < n_ne)
            def _():
                t = newin[pl.ds(i, _LANES)][0]
                bc = counts[pl.ds(t, _LANES)][0]
                erow = bucket[t, pl.ds(0, _BCAP)]

                def lab_body(q2, c):
                    e = _scalar(jnp.where(lane == q2, erow, 0))
                    j = e >> 12
                    q = e & 4095
                    jv = jnp.full((_LANES,), j, jnp.int32)
                    o = scell[0]
                    ob = (o & 127) * dim
                    for k in range(dim // _LANES):
                        cvec = lane + k * _LANES
                        vals = plsc.load_gather(win_v.at[slot], [cvec, jv])
                        rowbuf[pl.ds(ob + k * _LANES, _LANES)] = vals
                    vecb = own_b[pl.ds(q, _LANES)]
                    plsc.store_scatter(
                        own_b2, [jnp.full((_LANES,), o, jnp.int32)],
                        vecb, mask=lane0)
                    scell[0] = o + 1
                    return c

                lax.fori_loop(0, bc, lab_body, 0)

                o = scell[0]
                fl = scell[1]

                @pl.when(o - fl >= 64)
                def _():
                    half = ((fl >> 6) & 1) * (64 * dim)
                    dst = pl.multiple_of((wid * _CAP + fl) * dim, 64)
                    pltpu.sync_copy(rowbuf.at[pl.ds(half, 64 * dim)],
                                    rows_hbm.at[pl.ds(dst, 64 * dim)])
                    scell[1] = fl + 64

        scell[0] = jnp.int32(0)
        scell[1] = jnp.int32(0)

        for s in range(_DEPTH - 1):
            fetch(jnp.int32(s), s)

        n_blk = (tc_per_w + _DEPTH - 1) // _DEPTH

        def blk_body(blk, c):
            i0 = blk * _DEPTH
            for s in range(_DEPTH):
                i = i0 + s
                wait(i, s)
                extract(i, s)
                fetch(i + _DEPTH - 1, (s + _DEPTH - 1) % _DEPTH)
            return c

        lax.fori_loop(0, n_blk, blk_body, 0)

        o_real = scell[0]
        o_pad = (o_real + 63) & ~63

        @pl.when(o_real > 0)
        def _():
            lrow = (o_real - 1) & 127
            lastb = own_b2[pl.ds(o_real - 1, _LANES)]

            def pad_body(sl, c):
                for k in range(dim // _LANES):
                    rowbuf[pl.ds((sl & 127) * dim + k * _LANES, _LANES)] = (
                        rowbuf[pl.ds(lrow * dim + k * _LANES, _LANES)])
                plsc.store_scatter(
                    own_b2, [jnp.full((_LANES,), sl, jnp.int32)],
                    lastb, mask=lane0)
                return c

            lax.fori_loop(o_real, o_pad, pad_body, 0)
            scell[0] = o_pad

        cw = jnp.full((_LANES,), 0, jnp.int32)
        plsc.store_scatter(own_r, [lane], cw, mask=lane < 8)
        plsc.store_scatter(own_r, [lane], jnp.full((_LANES,), o_pad // 64,
                                                   jnp.int32), mask=lane0)
        pltpu.sync_copy(own_r.at[pl.ds(0, 8)],
                        cnt_hbm.at[pl.ds(wid * 8, 8)])

        def drain(d, c):
            o = scell[0]
            fl = scell[1]

            @pl.when(fl < o)
            def _():
                half = ((fl >> 6) & 1) * (64 * dim)
                dst = pl.multiple_of((wid * _CAP + fl) * dim, 64)
                pltpu.sync_copy(rowbuf.at[pl.ds(half, 64 * dim)],
                                rows_hbm.at[pl.ds(dst, 64 * dim)])
                scell[1] = fl + 64

            return c

        lax.fori_loop(0, _CAP // 64 + 1, drain, 0)

        pltpu.sync_copy(own_b2.at[pl.ds(0, _CAP)],
                        pos_hbm.at[pl.ds(wid * _CAP, _CAP)])

    rows_flat, b_list, cnts = scan_kernel(labels, table_t, tail_t)
    rows = rows_flat.reshape(nw * _CAP, dim)
    idx2 = b_list.reshape(nw * (_CAP // 64), 64)
    n_chunk = _CAP // 64

    @functools.partial(
        pl.kernel,
        mesh=mesh,
        out_type=jax.ShapeDtypeStruct((batch, dim), jnp.float32),
        compiler_params=pltpu.CompilerParams(use_tc_tiling_on_sc=False),
        scratch_types=[
            pltpu.VMEM((n_chunk, 64), jnp.int32),
            pltpu.VMEM((_LANES,), jnp.int32),
            pltpu.VMEM((_CAP, dim), jnp.float32),
            pltpu.SemaphoreType.DMA,
        ],
    )
    def permute_kernel(pos_hbm, rows_hbm, cnt_hbm, out_hbm,
                       idx_v, cbuf, rows_v, sem):
        wid = lax.axis_index("s") * info.num_cores + lax.axis_index("c")
        base = wid * _CAP
        pltpu.sync_copy(pos_hbm.at[pl.ds(wid * n_chunk, n_chunk), :], idx_v)
        pltpu.sync_copy(cnt_hbm.at[pl.ds(wid * 8, _LANES)], cbuf)
        nch = cbuf[pl.ds(0, _LANES)][0]
        pltpu.sync_copy(rows_hbm.at[pl.ds(base, _CAP), :], rows_v)
        for k in range(n_chunk):
            @pl.when(k < nch)
            def _():
                pltpu.async_copy(rows_v.at[pl.ds(k * 64, 64), :],
                                 out_hbm.at[idx_v.at[k]], sem)
        for k in range(n_chunk):
            @pl.when(k < nch)
            def _():
                pltpu.make_async_copy(rows_v.at[pl.ds(k * 64, 64), :],
                                      out_hbm.at[idx_v.at[k]], sem).wait()

    return permute_kernel(idx2, rows, cnts)


# speedup vs baseline: 1.6171x; 1.0011x over previous
import functools

import jax
import jax.numpy as jnp
from jax import lax
from jax.experimental import pallas as pl
from jax.experimental.pallas import tpu as pltpu
from jax.experimental.pallas import tpu_sc as plsc

_LANES = 16
_WIN = 128
_CAP = 640
_BCAP = 16
_DEPTH = 6


def _scalar(x):
    return lax.reduce_max(x, axes=(0,))


def kernel(labels, embedding_table):
    info = plsc.get_sparse_core_info()
    nw = info.num_cores * info.num_subcores
    (batch,) = labels.shape
    num_rows, dim = embedding_table.shape
    labels = labels.astype(jnp.int32)
    table_t = embedding_table.T

    n_tc = (num_rows + _WIN - 1) // _WIN
    tc_per_w = ((n_tc + nw - 1) // nw + 1) // 2 * 2
    last_tc = n_tc - 1
    last_w = num_rows - last_tc * _WIN
    tail_t = jnp.pad(
        lax.slice(table_t, (0, last_tc * _WIN), (dim, num_rows)),
        ((0, 0), (0, _WIN - last_w)))

    mesh = plsc.VectorSubcoreMesh(core_axis_name="c", subcore_axis_name="s")

    @functools.partial(
        pl.kernel,
        mesh=mesh,
        out_type=(
            jax.ShapeDtypeStruct((nw * _CAP * dim,), jnp.float32),
            jax.ShapeDtypeStruct((nw * _CAP,), jnp.int32),
            jax.ShapeDtypeStruct((nw * 8 + 8,), jnp.int32),
        ),
        compiler_params=pltpu.CompilerParams(needs_layout_passes=False),
        scratch_types=[
            pltpu.VMEM((1024,), jnp.int32),
            pltpu.VMEM((_CAP + _LANES,), jnp.int32),
            pltpu.VMEM((_CAP + _LANES,), jnp.int32),
            pltpu.VMEM((_CAP + _LANES,), jnp.int32),
            pltpu.VMEM((_DEPTH, dim, _WIN), jnp.float32),
            pltpu.VMEM((2 * 64 * dim,), jnp.float32),
            pltpu.VMEM((tc_per_w + 2, _BCAP), jnp.int32),
            pltpu.VMEM((tc_per_w + 2 + _LANES, ), jnp.int32),
            pltpu.VMEM((tc_per_w + 2 + _LANES, ), jnp.int32),
            pltpu.SMEM((8,), jnp.int32),
        ] + [pltpu.SemaphoreType.DMA] * _DEPTH,
    )
    def scan_kernel(idx_hbm, table_hbm, tail_hbm, rows_hbm, pos_hbm,
                    cnt_hbm, idx_v, own_r, own_b, own_b2, win_v, rowbuf,
                    bucket, counts, newin, scell, *sems):
        wid = lax.axis_index("s") * info.num_cores + lax.axis_index("c")
        tc0 = wid * tc_per_w
        lo_w = tc0 * _WIN
        hi_w = (tc0 + tc_per_w) * _WIN
        lane = lax.iota(jnp.int32, _LANES)
        zeros = jnp.zeros((_LANES,), jnp.int32)
        def zcount(k, c):
            counts[pl.ds(k * _LANES, _LANES)] = zeros
            return c

        lax.fori_loop(0, (tc_per_w + 2 + _LANES) // _LANES, zcount, 0)

        n_stage = batch // 512

        def stage_fetch(st, half):
            @pl.when(st < n_stage)
            def _():
                pltpu.async_copy(idx_hbm.at[pl.ds(st * 512, 512)],
                                 idx_v.at[pl.ds(half * 512, 512)],
                                 sems[half])

        def stage_wait(st, half):
            @pl.when(st < n_stage)
            def _():
                pltpu.make_async_copy(idx_hbm.at[pl.ds(st * 512, 512)],
                                      idx_v.at[pl.ds(half * 512, 512)],
                                      sems[half]).wait()

        def stage_bin(st, half, cnt):
            def bin_body(v, cnt):
                vec = idx_v[pl.ds(half * 512 + v * _LANES, _LANES)]
                m = (vec >= lo_w) & (vec < hi_w)
                bvec = lane + st * 512 + v * _LANES
                plsc.store_compressed(own_r.at[pl.ds(cnt, _LANES)], vec,
                                      mask=m)
                plsc.store_compressed(own_b.at[pl.ds(cnt, _LANES)], bvec,
                                      mask=m)
                k = plsc.all_reduce_population_count(m)[0]
                return jnp.minimum(cnt + k, _CAP)

            return lax.fori_loop(0, 512 // _LANES, bin_body, cnt)

        stage_fetch(jnp.int32(0), 0)

        def stagepair_body(sp, cnt):
            s0 = sp * 2
            stage_wait(s0, 0)
            stage_fetch(s0 + 1, 1)
            cnt = stage_bin(s0, 0, cnt)
            stage_wait(s0 + 1, 1)
            stage_fetch(s0 + 2, 0)
            cnt = stage_bin(s0 + 1, 1, cnt)
            return cnt

        cnt = lax.fori_loop(0, n_stage // 2, stagepair_body, jnp.int32(0))

        lane0 = lane == 0

        def buck_body(q, c):
            r = own_r[pl.ds(q, _LANES)][0]
            t = (r - lo_w) >> 7
            j = r & (_WIN - 1)
            bc = counts[pl.ds(t, _LANES)][0]

            @pl.when(bc < _BCAP)
            def _():
                e = (j << 12) | q
                plsc.store_scatter(
                    bucket,
                    [jnp.full((_LANES,), t, jnp.int32),
                     jnp.full((_LANES,), bc, jnp.int32)],
                    jnp.full((_LANES,), e, jnp.int32), mask=lane0)
                plsc.store_scatter(
                    counts, [jnp.full((_LANES,), t, jnp.int32)],
                    jnp.full((_LANES,), bc + 1, jnp.int32), mask=lane0)

            return c

        lax.fori_loop(0, cnt, buck_body, 0)

        def ne_body(t, n):
            bc = counts[pl.ds(t, _LANES)][0]

            @pl.when(bc > 0)
            def _():
                plsc.store_scatter(
                    newin, [jnp.full((_LANES,), n, jnp.int32)],
                    jnp.full((_LANES,), t, jnp.int32), mask=lane0)

            return n + jnp.where(bc > 0, 1, 0).astype(jnp.int32)

        n_ne = lax.fori_loop(0, tc_per_w, ne_body, jnp.int32(0))

        def fetch(i, slot):
            @pl.when(i < n_ne)
            def _():
                t = newin[pl.ds(i, _LANES)][0]
                tc = tc0 + t

                @pl.when(tc < last_tc)
                def _():
                    off = pl.multiple_of(tc * _WIN, _WIN)
                    pltpu.async_copy(table_hbm.at[:, pl.ds(off, _WIN)],
                                     win_v.at[slot], sems[slot])

                @pl.when(tc == last_tc)
                def _():
                    pltpu.async_copy(tail_hbm, win_v.at[slot], sems[slot])

        def wait(i, slot):
            @pl.when(i < n_ne)
            def _():
                t = newin[pl.ds(i, _LANES)][0]
                tc = tc0 + t

                @pl.when(tc < last_tc)
                def _():
                    off = pl.multiple_of(tc * _WIN, _WIN)
                    pltpu.make_async_copy(table_hbm.at[:, pl.ds(off, _WIN)],
                                          win_v.at[slot], sems[slot]).wait()

                @pl.when(tc == last_tc)
                def _():
                    pltpu.make_async_copy(tail_hbm, win_v.at[slot],
                                          sems[slot]).wait()

        def extract(i, slot):
            @pl.when(i < n_ne)
            def _():
                t = newin[pl.ds(i, _LANES)][0]
                bc = counts[pl.ds(t, _LANES)][0]
                erow = bucket[t, pl.ds(0, _BCAP)]

                def lab_body(q2, c):
                    e = _scalar(jnp.where(lane == q2, erow, 0))
                    j = e >> 12
                    q = e & 4095
                    jv = jnp.full((_LANES,), j, jnp.int32)
                    o = scell[0]
                    ob = (o & 127) * dim
                    for k in range(dim // _LANES):
                        cvec = lane + k * _LANES
                        vals = plsc.load_gather(win_v.at[slot], [cvec, jv])
                        rowbuf[pl.ds(ob + k * _LANES, _LANES)] = vals
                    vecb = own_b[pl.ds(q, _LANES)]
                    plsc.store_scatter(
                        own_b2, [jnp.full((_LANES,), o, jnp.int32)],
                        vecb, mask=lane0)
                    scell[0] = o + 1
                    return c

                lax.fori_loop(0, bc, lab_body, 0)

                o = scell[0]
                fl = scell[1]

                @pl.when(o - fl >= 64)
                def _():
                    half = ((fl >> 6) & 1) * (64 * dim)
                    dst = pl.multiple_of((wid * _CAP + fl) * dim, 64)
                    pltpu.sync_copy(rowbuf.at[pl.ds(half, 64 * dim)],
                                    rows_hbm.at[pl.ds(dst, 64 * dim)])
                    scell[1] = fl + 64

        scell[0] = jnp.int32(0)
        scell[1] = jnp.int32(0)

        for s in range(_DEPTH - 1):
            fetch(jnp.int32(s), s)

        n_blk = (tc_per_w + _DEPTH - 1) // _DEPTH

        def blk_body(blk, c):
            i0 = blk * _DEPTH
            for s in range(_DEPTH):
                i = i0 + s
                wait(i, s)
                extract(i, s)
                fetch(i + _DEPTH - 1, (s + _DEPTH - 1) % _DEPTH)
            return c

        lax.fori_loop(0, n_blk, blk_body, 0)

        o_real = scell[0]
        o_pad = (o_real + 63) & ~63

        @pl.when(o_real > 0)
        def _():
            lrow = (o_real - 1) & 127
            lastb = own_b2[pl.ds(o_real - 1, _LANES)]

            def pad_body(sl, c):
                for k in range(dim // _LANES):
                    rowbuf[pl.ds((sl & 127) * dim + k * _LANES, _LANES)] = (
                        rowbuf[pl.ds(lrow * dim + k * _LANES, _LANES)])
                plsc.store_scatter(
                    own_b2, [jnp.full((_LANES,), sl, jnp.int32)],
                    lastb, mask=lane0)
                return c

            lax.fori_loop(o_real, o_pad, pad_body, 0)
            scell[0] = o_pad

        cw = jnp.full((_LANES,), 0, jnp.int32)
        plsc.store_scatter(own_r, [lane], cw, mask=lane < 8)
        plsc.store_scatter(own_r, [lane], jnp.full((_LANES,), o_pad // 64,
                                                   jnp.int32), mask=lane0)
        pltpu.sync_copy(own_r.at[pl.ds(0, 8)],
                        cnt_hbm.at[pl.ds(wid * 8, 8)])

        def drain(d, c):
            o = scell[0]
            fl = scell[1]

            @pl.when(fl < o)
            def _():
                half = ((fl >> 6) & 1) * (64 * dim)
                dst = pl.multiple_of((wid * _CAP + fl) * dim, 64)
                pltpu.sync_copy(rowbuf.at[pl.ds(half, 64 * dim)],
                                rows_hbm.at[pl.ds(dst, 64 * dim)])
                scell[1] = fl + 64

            return c

        lax.fori_loop(0, _CAP // 64 + 1, drain, 0)

        pltpu.sync_copy(own_b2.at[pl.ds(0, _CAP)],
                        pos_hbm.at[pl.ds(wid * _CAP, _CAP)])

    rows_flat, b_list, cnts = scan_kernel(labels, table_t, tail_t)
    rows = rows_flat.reshape(nw * _CAP, dim)
    idx2 = b_list.reshape(nw * (_CAP // 64), 64)
    n_chunk = _CAP // 64

    @functools.partial(
        pl.kernel,
        mesh=mesh,
        out_type=jax.ShapeDtypeStruct((batch, dim), jnp.float32),
        compiler_params=pltpu.CompilerParams(use_tc_tiling_on_sc=False),
        scratch_types=[
            pltpu.VMEM((n_chunk, 64), jnp.int32),
            pltpu.VMEM((_LANES,), jnp.int32),
            pltpu.VMEM((_CAP, dim), jnp.float32),
            pltpu.SemaphoreType.DMA,
        ],
    )
    def permute_kernel(pos_hbm, rows_hbm, cnt_hbm, out_hbm,
                       idx_v, cbuf, rows_v, sem):
        wid = lax.axis_index("s") * info.num_cores + lax.axis_index("c")
        base = wid * _CAP
        pltpu.sync_copy(pos_hbm.at[pl.ds(wid * n_chunk, n_chunk), :], idx_v)
        pltpu.sync_copy(cnt_hbm.at[pl.ds(wid * 8, _LANES)], cbuf)
        nch = cbuf[pl.ds(0, _LANES)][0]
        pltpu.sync_copy(rows_hbm.at[pl.ds(base, _CAP), :], rows_v)
        for k in range(n_chunk):
            @pl.when(k < nch)
            def _():
                pltpu.async_copy(rows_v.at[pl.ds(k * 64, 64), :],
                                 out_hbm.at[idx_v.at[k]], sem)
        for k in range(n_chunk):
            @pl.when(k < nch)
            def _():
                pltpu.make_async_copy(rows_v.at[pl.ds(k * 64, 64), :],
                                      out_hbm.at[idx_v.at[k]], sem).wait()

    return permute_kernel(idx2, rows, cnts)
